# Initial kernel scaffold; baseline (speedup 1.0000x reference)
import jax
import jax.numpy as jnp
from jax.experimental import pallas as pl


def kernel(x, edge_index, edge_attr, ew1, eb1, ew2, eb2, nw1, nb1, nw2, nb2, eln_w, eln_b, nln_w, nln_b):
    raise NotImplementedError("write your pallas kernel here")


# speedup vs baseline: 2.1808x; 2.1808x over previous
import functools

import jax
import jax.numpy as jnp
from jax import lax
from jax.experimental import pallas as pl
from jax.experimental.pallas import tpu as pltpu
from jax.experimental.pallas import tpu_sc as plsc

N = 10000
E = 320000
ND = 128
ED = 16
H = 512
EPS = 1e-5

NC = 2
NS = 16
NW = NC * NS
LW = 128
IR = E // LW
CR = 5120
RT = CR + NS
EB = 64

BE = 2000
BN = 1000


def _gather_body(x_hbm, sidx_hbm, ridx_hbm, xs_hbm, xr_hbm, idx_v, rows_v, sem):
    c = lax.axis_index("c")
    s = lax.axis_index("s")
    wid = s * NC + c
    lo = wid * IR // NW
    hi = (wid + 1) * IR // NW

    def body(j, carry):
        pltpu.sync_copy(sidx_hbm.at[j], idx_v)
        pltpu.async_copy(x_hbm.at[idx_v], rows_v, sem).wait()
        pltpu.sync_copy(rows_v, xs_hbm.at[pl.ds(j * LW, LW)])
        pltpu.sync_copy(ridx_hbm.at[j], idx_v)
        pltpu.async_copy(x_hbm.at[idx_v], rows_v, sem).wait()
        pltpu.sync_copy(rows_v, xr_hbm.at[pl.ds(j * LW, LW)])
        return carry

    lax.fori_loop(lo, hi, body, 0)


@functools.cache
def _make_gather():
    return pl.kernel(
        _gather_body,
        out_type=(
            jax.ShapeDtypeStruct((E, ND), jnp.float32),
            jax.ShapeDtypeStruct((E, ND), jnp.float32),
        ),
        mesh=plsc.VectorSubcoreMesh(
            core_axis_name="c", subcore_axis_name="s", num_cores=NC, num_subcores=NS
        ),
        scratch_types=[
            pltpu.VMEM((LW,), jnp.int32),
            pltpu.VMEM((LW, ND), jnp.float32),
            pltpu.SemaphoreType.DMA,
        ],
    )


def _gather(x, senders, receivers):
    return _make_gather()(x, senders, receivers)


def _edge_mlp_body(xs, xr, ea, w1a, w1b, w1c, b1, w2, b2, pay, stats):
    pre = (
        jnp.dot(xs[...], w1a[...], preferred_element_type=jnp.float32)
        + jnp.dot(xr[...], w1b[...], preferred_element_type=jnp.float32)
        + jnp.dot(ea[...], w1c[...], preferred_element_type=jnp.float32)
        + b1[...]
    )
    h = pre * jax.nn.sigmoid(pre)
    u = jnp.dot(h, w2[...], preferred_element_type=jnp.float32) + b2[...]
    pay[...] = jnp.concatenate(
        [u, jnp.ones((BE, 1), jnp.float32), jnp.zeros((BE, 127 - ED), jnp.float32)],
        axis=1,
    )
    ne = ea[...] + u
    s1 = jnp.sum(ne)
    s2 = jnp.sum(ne * ne)
    lane = lax.broadcasted_iota(jnp.int32, (1, 128), 1)
    vec = jnp.where(lane == 0, s1, 0.0) + jnp.where(lane == 1, s2, 0.0)

    @pl.when(pl.program_id(0) == 0)
    def _():
        stats[...] = jnp.zeros_like(stats)

    stats[...] += vec


def _edge_mlp(xs, xr, ea, w1a, w1b, w1c, b1, w2, b2):
    return pl.pallas_call(
        _edge_mlp_body,
        grid=(E // BE,),
        in_specs=[
            pl.BlockSpec((BE, ND), lambda i: (i, 0)),
            pl.BlockSpec((BE, ND), lambda i: (i, 0)),
            pl.BlockSpec((BE, ED), lambda i: (i, 0)),
            pl.BlockSpec((ND, H), lambda i: (0, 0)),
            pl.BlockSpec((ND, H), lambda i: (0, 0)),
            pl.BlockSpec((ED, H), lambda i: (0, 0)),
            pl.BlockSpec((1, H), lambda i: (0, 0)),
            pl.BlockSpec((H, ED), lambda i: (0, 0)),
            pl.BlockSpec((1, ED), lambda i: (0, 0)),
        ],
        out_specs=[
            pl.BlockSpec((BE, 128), lambda i: (i, 0)),
            pl.BlockSpec((1, 128), lambda i: (0, 0)),
        ],
        out_shape=[
            jax.ShapeDtypeStruct((E, 128), jnp.float32),
            jax.ShapeDtypeStruct((1, 128), jnp.float32),
        ],
    )(xs, xr, ea, w1a, w1b, w1c, b1, w2, b2)


def _scatter_body(pay_hbm, ridx_hbm, zeros_hbm, seg_hbm,
                  idx_v, cidx_v, pay_v, ebuf_v, seg_sh):
    c = lax.axis_index("c")
    s = lax.axis_index("s")
    base = c * CR
    trash = CR + s

    @pl.when(s == 0)
    def _():
        pltpu.sync_copy(zeros_hbm, seg_sh)

    plsc.subcore_barrier()

    lo = s * IR // NS
    hi = (s + 1) * IR // NS

    def body(j, carry):
        pltpu.sync_copy(ridx_hbm.at[j], idx_v)
        pltpu.sync_copy(pay_hbm.at[pl.ds(j * LW, LW)], pay_v)
        for k in range(LW // 16):
            v = idx_v[pl.ds(k * 16, 16)]
            loc = v - base
            ok = (loc >= 0) & (loc < CR)
            cidx_v[pl.ds(k * 16, 16)] = jnp.where(ok, loc, trash)
        pltpu.sync_copy(pay_v, seg_sh.at[cidx_v], add=True)
        return carry

    lax.fori_loop(lo, hi, body, 0)
    plsc.subcore_barrier()

    def ebody(k, carry):
        b = s * (CR // NS) + k * EB
        pltpu.sync_copy(seg_sh.at[pl.ds(b, EB)], ebuf_v)
        pltpu.sync_copy(ebuf_v, seg_hbm.at[c, pl.ds(b, EB)])
        return carry

    lax.fori_loop(0, CR // NS // EB, ebody, 0)


@functools.cache
def _make_scatter():
    return pl.kernel(
        _scatter_body,
        out_type=jax.ShapeDtypeStruct((NC, CR, 128), jnp.float32),
        mesh=plsc.VectorSubcoreMesh(
            core_axis_name="c", subcore_axis_name="s", num_cores=NC, num_subcores=NS
        ),
        scratch_types=[
            pltpu.VMEM((LW,), jnp.int32),
            pltpu.VMEM((LW,), jnp.int32),
            pltpu.VMEM((LW, 128), jnp.float32),
            pltpu.VMEM((EB, 128), jnp.float32),
            pltpu.VMEM_SHARED((RT, 128), jnp.float32),
        ],
    )


def _scatter(pay, receivers, zeros_init):
    return _make_scatter()(pay, receivers, zeros_init)


def _edge_norm_body(ea, pay, stats, w, b, out):
    st = stats[...]
    lane = lax.broadcasted_iota(jnp.int32, (1, 128), 1)
    tot = jnp.float32(E * ED)
    s1 = jnp.sum(jnp.where(lane == 0, st, 0.0))
    s2 = jnp.sum(jnp.where(lane == 1, st, 0.0))
    mean = s1 / tot
    var = jnp.maximum(s2 / tot - mean * mean, 0.0)
    inv = 1.0 / (jnp.sqrt(var) + EPS)
    ne = ea[...] + pay[:, :ED]
    out[...] = (ne - mean) * inv * w[...] + b[...]


def _edge_norm(ea, pay, stats, w, b):
    return pl.pallas_call(
        _edge_norm_body,
        grid=(E // BE,),
        in_specs=[
            pl.BlockSpec((BE, ED), lambda i: (i, 0)),
            pl.BlockSpec((BE, 128), lambda i: (i, 0)),
            pl.BlockSpec((1, 128), lambda i: (0, 0)),
            pl.BlockSpec((1, ED), lambda i: (0, 0)),
            pl.BlockSpec((1, ED), lambda i: (0, 0)),
        ],
        out_specs=pl.BlockSpec((BE, ED), lambda i: (i, 0)),
        out_shape=jax.ShapeDtypeStruct((E, ED), jnp.float32),
    )(ea, pay, stats, w, b)


def _node_body(x, segp, w1a, w1b, b1, w2, b2, lw, lb, out):
    sp = segp[...]
    agg = sp[:, :ED] / jnp.maximum(sp[:, ED:ED + 1], 1.0)
    pre = (
        jnp.dot(x[...], w1a[...], preferred_element_type=jnp.float32)
        + jnp.dot(agg, w1b[...], preferred_element_type=jnp.float32)
        + b1[...]
    )
    g = pre * jax.nn.sigmoid(pre)
    u = jnp.dot(g, w2[...], preferred_element_type=jnp.float32) + b2[...]
    nx = x[...] + u
    mu = jnp.mean(nx, axis=-1, keepdims=True)
    d = nx - mu
    var = jnp.mean(d * d, axis=-1, keepdims=True)
    out[...] = d * lax.rsqrt(var + EPS) * lw[...] + lb[...]


def _node_mlp(x, segp, w1a, w1b, b1, w2, b2, lw, lb):
    return pl.pallas_call(
        _node_body,
        grid=(N // BN,),
        in_specs=[
            pl.BlockSpec((BN, ND), lambda i: (i, 0)),
            pl.BlockSpec((BN, 128), lambda i: (i, 0)),
            pl.BlockSpec((ND, H), lambda i: (0, 0)),
            pl.BlockSpec((ED, H), lambda i: (0, 0)),
            pl.BlockSpec((1, H), lambda i: (0, 0)),
            pl.BlockSpec((H, ND), lambda i: (0, 0)),
            pl.BlockSpec((1, ND), lambda i: (0, 0)),
            pl.BlockSpec((1, ND), lambda i: (0, 0)),
            pl.BlockSpec((1, ND), lambda i: (0, 0)),
        ],
        out_specs=pl.BlockSpec((BN, ND), lambda i: (i, 0)),
        out_shape=jax.ShapeDtypeStruct((N, ND), jnp.float32),
    )(x, segp, w1a, w1b, b1, w2, b2, lw, lb)


def kernel(x, edge_index, edge_attr, ew1, eb1, ew2, eb2, nw1, nb1, nw2, nb2,
           eln_w, eln_b, nln_w, nln_b):
    senders = edge_index[0].reshape(IR, LW)
    receivers = edge_index[1].reshape(IR, LW)

    xs, xr = _gather(x, senders, receivers)

    pay, stats = _edge_mlp(
        xs, xr, edge_attr,
        ew1[:ND], ew1[ND:2 * ND], ew1[2 * ND:],
        eb1.reshape(1, H), ew2, eb2.reshape(1, ED),
    )

    zeros_init = jnp.zeros((RT, 128), dtype=jnp.float32)
    seg = _scatter(pay, receivers, zeros_init)
    segp = jnp.concatenate([seg[0], seg[1]], axis=0)[:N]

    new_edge_attr = _edge_norm(
        edge_attr, pay, stats, eln_w.reshape(1, ED), eln_b.reshape(1, ED)
    )

    new_x = _node_mlp(
        x, segp,
        nw1[:ND], nw1[ND:], nb1.reshape(1, H), nw2, nb2.reshape(1, ND),
        nln_w.reshape(1, ND), nln_b.reshape(1, ND),
    )
    return new_x, new_edge_attr
